# baseline (speedup 1.0000x reference)
import functools

import jax
import jax.numpy as jnp
from jax import lax
from jax.experimental import pallas as pl
from jax.experimental.pallas import tpu as pltpu
from jax.experimental.pallas import tpu_sc as plsc

_N_CHAINS = 64
_N_STATES = 16
_FLAT = 2048 * 16
_NW = 32
_CPW = _N_CHAINS // _NW


def _gibbs_body(x_hbm, w_hbm, g_hbm, o_hbm, xv, wv, gv, sem):
    nc = lax.axis_size("c")
    wid = lax.axis_index("s") * nc + lax.axis_index("c")
    base = wid * (_CPW * _FLAT)
    big = pltpu.make_async_copy(x_hbm.at[pl.ds(base, _CPW * _FLAT)], xv, sem)
    big.start()
    pltpu.sync_copy(w_hbm, wv)
    pltpu.sync_copy(g_hbm.at[pl.ds(wid * _CPW * _N_STATES,
                                   _CPW * _N_STATES)], gv)
    lw = wv[...]
    iota = lax.iota(jnp.int32, _N_STATES)
    big.wait()
    for j in range(_CPW):
        lv = lw + gv[pl.ds(j * _N_STATES, _N_STATES)]
        m = jnp.max(lv)
        sel = jnp.min(jnp.where(lv == m, iota, _N_STATES))
        xv[pl.ds(j * _FLAT, _N_STATES)] = (iota == sel).astype(lw.dtype)
    pltpu.sync_copy(xv, o_hbm.at[pl.ds(base, _CPW * _FLAT)])


def kernel(x, W):
    n_chains, n_dims, n_states = x.shape
    x1 = x.reshape(-1)
    w16 = W[:n_states]
    g = jax.random.gumbel(jax.random.key(42), (n_chains, n_states),
                          dtype=x.dtype).reshape(-1)
    mesh = plsc.VectorSubcoreMesh(core_axis_name="c", subcore_axis_name="s")
    run = functools.partial(
        pl.kernel,
        out_type=jax.ShapeDtypeStruct((n_chains * n_dims * n_states,),
                                      x.dtype),
        mesh=mesh,
        compiler_params=pltpu.CompilerParams(needs_layout_passes=False),
        scratch_types=[
            pltpu.VMEM((_CPW * _FLAT,), x.dtype),
            pltpu.VMEM((_N_STATES,), x.dtype),
            pltpu.VMEM((_CPW * _N_STATES,), x.dtype),
            pltpu.SemaphoreType.DMA,
        ],
    )(_gibbs_body)
    out = run(x1, w16, g)
    return out.reshape(n_chains, n_dims, n_states)


# speedup vs baseline: 1.1107x; 1.1107x over previous
import jax
import jax.numpy as jnp
from jax.experimental import pallas as pl

_N_STATES = 16
_CB = 8


def _gibbs_body(x_ref, w16_ref, g_ref, o_ref):
    xv = x_ref[...]
    logits = w16_ref[...] + g_ref[...]
    m = jnp.max(logits, axis=1, keepdims=True)
    iota = jax.lax.broadcasted_iota(jnp.int32, (_CB, _N_STATES), 1)
    sel = jnp.min(jnp.where(logits == m, iota, _N_STATES), axis=1,
                  keepdims=True)
    o_ref[...] = xv
    o_ref[:, 0, :] = (iota == sel).astype(xv.dtype)


def kernel(x, W):
    n_chains, n_dims, n_states = x.shape
    w16 = W[:n_states].reshape(1, n_states)
    g = jax.random.gumbel(jax.random.key(42), (n_chains, n_states),
                          dtype=x.dtype)
    return pl.pallas_call(
        _gibbs_body,
        grid=(n_chains // _CB,),
        in_specs=[
            pl.BlockSpec((_CB, n_dims, n_states), lambda i: (i, 0, 0)),
            pl.BlockSpec((1, n_states), lambda i: (0, 0)),
            pl.BlockSpec((_CB, n_states), lambda i: (i, 0)),
        ],
        out_specs=pl.BlockSpec((_CB, n_dims, n_states), lambda i: (i, 0, 0)),
        out_shape=jax.ShapeDtypeStruct((n_chains, n_dims, n_states), x.dtype),
    )(x, w16, g)


# speedup vs baseline: 1.8055x; 1.6255x over previous
import jax
import jax.numpy as jnp
from jax.experimental import pallas as pl
from jax.experimental.pallas import tpu as pltpu

_N_STATES = 16
_CB = 8


def _gibbs_body(x_ref, w16_ref, g_ref, o_ref):
    xv = x_ref[...]
    logits = w16_ref[...] + g_ref[...]
    m = jnp.max(logits, axis=1, keepdims=True)
    iota = jax.lax.broadcasted_iota(jnp.int32, (_CB, _N_STATES), 1)
    sel = jnp.min(jnp.where(logits == m, iota, _N_STATES), axis=1,
                  keepdims=True)
    o_ref[...] = xv
    o_ref[:, :_N_STATES] = (iota == sel).astype(xv.dtype)


def kernel(x, W):
    n_chains, n_dims, n_states = x.shape
    flat = n_dims * n_states
    x2 = x.reshape(n_chains, flat)
    w16 = W[:n_states].reshape(1, n_states)
    g = jax.random.gumbel(jax.random.key(42), (n_chains, n_states),
                          dtype=x.dtype)
    out = pl.pallas_call(
        _gibbs_body,
        grid=(n_chains // _CB,),
        in_specs=[
            pl.BlockSpec((_CB, flat), lambda i: (i, 0)),
            pl.BlockSpec((1, n_states), lambda i: (0, 0)),
            pl.BlockSpec((_CB, n_states), lambda i: (i, 0)),
        ],
        out_specs=pl.BlockSpec((_CB, flat), lambda i: (i, 0)),
        out_shape=jax.ShapeDtypeStruct((n_chains, flat), x.dtype),
    )(x2, w16, g)
    return out.reshape(n_chains, n_dims, n_states)


# speedup vs baseline: 10.4569x; 5.7917x over previous
import jax
import jax.numpy as jnp
from jax.experimental import pallas as pl

_N_STATES = 16
_CB = 8


def _gibbs_body(x_ref, w16_ref, g_ref, o_ref):
    xv = x_ref[...]
    n_dims = xv.shape[2]
    logits = w16_ref[...] + g_ref[...]
    m = jnp.max(logits, axis=1, keepdims=True)
    iota = jax.lax.broadcasted_iota(jnp.int32, (_CB, _N_STATES), 1)
    sel = jnp.min(jnp.where(logits == m, iota, _N_STATES), axis=1,
                  keepdims=True)
    onehot = (iota == sel).astype(xv.dtype)
    lane = jax.lax.broadcasted_iota(jnp.int32, (_CB, _N_STATES, n_dims), 2)
    o_ref[...] = jnp.where(lane == 0, onehot[:, :, None], xv)


def kernel(x, W):
    n_chains, n_dims, n_states = x.shape
    xt = jnp.transpose(x, (0, 2, 1))
    w16 = W[:n_states].reshape(1, n_states)
    g = jax.random.gumbel(jax.random.key(42), (n_chains, n_states),
                          dtype=x.dtype)
    ot = pl.pallas_call(
        _gibbs_body,
        grid=(n_chains // _CB,),
        in_specs=[
            pl.BlockSpec((_CB, n_states, n_dims), lambda i: (i, 0, 0)),
            pl.BlockSpec((1, n_states), lambda i: (0, 0)),
            pl.BlockSpec((_CB, n_states), lambda i: (i, 0)),
        ],
        out_specs=pl.BlockSpec((_CB, n_states, n_dims), lambda i: (i, 0, 0)),
        out_shape=jax.ShapeDtypeStruct((n_chains, n_states, n_dims), x.dtype),
    )(xt, w16, g)
    return jnp.transpose(ot, (0, 2, 1))


# speedup vs baseline: 13.1365x; 1.2562x over previous
import jax
import jax.numpy as jnp
from jax.experimental import pallas as pl

_N_STATES = 16
_CB = 16


def _gibbs_body(x_ref, w16_ref, g_ref, o_ref):
    xv = x_ref[...]
    n_dims = xv.shape[2]
    logits = w16_ref[...] + g_ref[...]
    m = jnp.max(logits, axis=1, keepdims=True)
    iota = jax.lax.broadcasted_iota(jnp.int32, (_CB, _N_STATES), 1)
    sel = jnp.min(jnp.where(logits == m, iota, _N_STATES), axis=1,
                  keepdims=True)
    onehot = (iota == sel).astype(xv.dtype)
    lane = jax.lax.broadcasted_iota(jnp.int32, (_CB, _N_STATES, n_dims), 2)
    o_ref[...] = jnp.where(lane == 0, onehot[:, :, None], xv)


def kernel(x, W):
    n_chains, n_dims, n_states = x.shape
    xt = jnp.transpose(x, (0, 2, 1))
    w16 = W[:n_states].reshape(1, n_states)
    g = jax.random.gumbel(jax.random.key(42), (n_chains, n_states),
                          dtype=x.dtype)
    ot = pl.pallas_call(
        _gibbs_body,
        grid=(n_chains // _CB,),
        in_specs=[
            pl.BlockSpec((_CB, n_states, n_dims), lambda i: (i, 0, 0)),
            pl.BlockSpec((1, n_states), lambda i: (0, 0)),
            pl.BlockSpec((_CB, n_states), lambda i: (i, 0)),
        ],
        out_specs=pl.BlockSpec((_CB, n_states, n_dims), lambda i: (i, 0, 0)),
        out_shape=jax.ShapeDtypeStruct((n_chains, n_states, n_dims), x.dtype),
    )(xt, w16, g)
    return jnp.transpose(ot, (0, 2, 1))


# speedup vs baseline: 14.9630x; 1.1390x over previous
import jax
import jax.numpy as jnp
from jax.experimental import pallas as pl

_N_STATES = 16
_CB = 32


def _gibbs_body(x_ref, w16_ref, g_ref, o_ref):
    xv = x_ref[...]
    n_dims = xv.shape[2]
    logits = w16_ref[...] + g_ref[...]
    m = jnp.max(logits, axis=1, keepdims=True)
    iota = jax.lax.broadcasted_iota(jnp.int32, (_CB, _N_STATES), 1)
    sel = jnp.min(jnp.where(logits == m, iota, _N_STATES), axis=1,
                  keepdims=True)
    onehot = (iota == sel).astype(xv.dtype)
    lane = jax.lax.broadcasted_iota(jnp.int32, (_CB, _N_STATES, n_dims), 2)
    o_ref[...] = jnp.where(lane == 0, onehot[:, :, None], xv)


def kernel(x, W):
    n_chains, n_dims, n_states = x.shape
    xt = jnp.transpose(x, (0, 2, 1))
    w16 = W[:n_states].reshape(1, n_states)
    g = jax.random.gumbel(jax.random.key(42), (n_chains, n_states),
                          dtype=x.dtype)
    ot = pl.pallas_call(
        _gibbs_body,
        grid=(n_chains // _CB,),
        in_specs=[
            pl.BlockSpec((_CB, n_states, n_dims), lambda i: (i, 0, 0)),
            pl.BlockSpec((1, n_states), lambda i: (0, 0)),
            pl.BlockSpec((_CB, n_states), lambda i: (i, 0)),
        ],
        out_specs=pl.BlockSpec((_CB, n_states, n_dims), lambda i: (i, 0, 0)),
        out_shape=jax.ShapeDtypeStruct((n_chains, n_states, n_dims), x.dtype),
    )(xt, w16, g)
    return jnp.transpose(ot, (0, 2, 1))
